# baseline (speedup 1.0000x reference)
import functools

import jax
import jax.numpy as jnp
from jax import lax
from jax.experimental import pallas as pl
from jax.experimental.pallas import tpu as pltpu
from jax.experimental.pallas import tpu_sc as plsc

NC = 2
NS = 16
NW = NC * NS
CH = 128


def _sc_deg(dst2d, zeros1, np_, nps):
    rpw = dst2d.shape[0] // NW
    mesh = plsc.VectorSubcoreMesh(core_axis_name="c", subcore_axis_name="s")

    @functools.partial(
        pl.kernel,
        out_type=jax.ShapeDtypeStruct((NC, NS, nps), jnp.float32),
        mesh=mesh,
        scratch_types=[
            pltpu.VMEM((rpw, CH), jnp.int32),
            pltpu.VMEM((CH,), jnp.float32),
            pltpu.VMEM_SHARED((np_,), jnp.float32),
        ],
    )
    def k(dst_hbm, z_hbm, out_hbm, idx_v, ones_v, deg_sh):
        cid = lax.axis_index("c")
        sid = lax.axis_index("s")
        wid = sid * NC + cid
        pltpu.sync_copy(z_hbm, deg_sh.at[pl.ds(sid * nps, nps)])

        @pl.loop(0, CH, step=16)
        def _(i):
            ones_v[pl.ds(i, 16)] = jnp.ones((16,), jnp.float32)

        pltpu.sync_copy(dst_hbm.at[pl.ds(wid * rpw, rpw)], idx_v)
        plsc.subcore_barrier()

        @pl.loop(0, rpw)
        def _(j):
            pltpu.sync_copy(ones_v, deg_sh.at[idx_v.at[j]], add=True)

        plsc.subcore_barrier()
        pltpu.sync_copy(deg_sh.at[pl.ds(sid * nps, nps)], out_hbm.at[cid, sid])

    return k(dst2d, zeros1)


CB = 48


def _sc_agg(h, src2d, dst2d, zeros2, np_, nps, b0):
    tblk = src2d.shape[0]
    tot = tblk // NS
    b1 = tot - b0
    d = h.shape[1]
    assert b0 % 8 == 0 and b1 % 8 == 0 and CB % 8 == 0
    mesh = plsc.VectorSubcoreMesh(core_axis_name="c", subcore_axis_name="s")

    nbuf = 2

    @functools.partial(
        pl.kernel,
        out_type=jax.ShapeDtypeStruct((NC, NS, nps, d), jnp.float32),
        mesh=mesh,
        scratch_types=[
            pltpu.VMEM((CB, CH), jnp.int32),
            pltpu.VMEM((CB, CH), jnp.int32),
            pltpu.VMEM((CH, d), jnp.float32),
            pltpu.VMEM((CH, d), jnp.float32),
            pltpu.VMEM_SHARED((np_, d), jnp.float32),
            pltpu.SemaphoreType.DMA,
            pltpu.SemaphoreType.DMA,
        ],
    )
    def k(h_hbm, s_hbm, d_hbm, z_hbm, out_hbm, src_v, dst_v, r0, r1,
          acc_sh, sem0, sem1):
        sems = [sem0, sem1]
        rows = [r0, r1]
        cid = lax.axis_index("c")
        sid = lax.axis_index("s")
        with jax.named_scope("agg_zero"):
            pltpu.sync_copy(z_hbm, acc_sh.at[pl.ds(sid * nps, nps)])
            plsc.subcore_barrier()

        def run(nblk, base):
            done = 0
            while done < nblk:
                nb = min(CB, nblk - done)
                cbase = base + done
                done += nb
                pltpu.sync_copy(s_hbm.at[pl.ds(cbase, nb)], src_v.at[pl.ds(0, nb)])
                pltpu.sync_copy(d_hbm.at[pl.ds(cbase, nb)], dst_v.at[pl.ds(0, nb)])

                for b in range(nbuf):
                    pltpu.async_copy(h_hbm.at[src_v.at[b]], rows[b], sems[b])

                @pl.loop(0, nb - nbuf, step=nbuf)
                def _(j):
                    for b in range(nbuf):
                        jj = j + b
                        pltpu.make_async_copy(
                            h_hbm.at[src_v.at[jj]], rows[b], sems[b]).wait()
                        pltpu.sync_copy(rows[b], acc_sh.at[dst_v.at[jj]],
                                        add=True)
                        pltpu.async_copy(
                            h_hbm.at[src_v.at[jj + nbuf]], rows[b], sems[b])

                for b in range(nbuf):
                    jj = nb - nbuf + b
                    pltpu.make_async_copy(
                        h_hbm.at[src_v.at[jj]], rows[b], sems[b]).wait()
                    pltpu.sync_copy(rows[b], acc_sh.at[dst_v.at[jj]],
                                    add=True)

        with jax.named_scope("agg_edges"):
            @pl.when(cid == 0)
            def _():
                run(b0, sid * b0)

            @pl.when(cid == 1)
            def _():
                run(b1, NS * b0 + sid * b1)

            plsc.subcore_barrier()

        with jax.named_scope("agg_dump"):
            pltpu.sync_copy(acc_sh.at[pl.ds(sid * nps, nps)],
                            out_hbm.at[cid, sid])

    return k(h, src2d, dst2d, zeros2)


def _tc_scale1(x, w1, degp, br):
    n, d = x.shape

    def body(x_ref, w_ref, degp_ref, ht_ref, dinv_ref):
        h = jnp.dot(x_ref[...], w_ref[...], preferred_element_type=jnp.float32)
        deg = degp_ref[0] + degp_ref[1] + 1.0
        dinv = lax.rsqrt(deg)
        dinv_ref[...] = dinv
        ht_ref[...] = h * dinv

    return pl.pallas_call(
        body,
        grid=(n // br,),
        in_specs=[
            pl.BlockSpec((br, d), lambda i: (i, 0)),
            pl.BlockSpec((d, d), lambda i: (0, 0)),
            pl.BlockSpec((NC, br, 1), lambda i: (0, i, 0)),
        ],
        out_specs=[
            pl.BlockSpec((br, d), lambda i: (i, 0)),
            pl.BlockSpec((br, 1), lambda i: (i, 0)),
        ],
        out_shape=[
            jax.ShapeDtypeStruct((n, d), jnp.float32),
            jax.ShapeDtypeStruct((n, 1), jnp.float32),
        ],
    )(x, w1, degp)


def _tc_mid(accp, ht1, dinv, b1, w2, br):
    n, d = ht1.shape

    def body(accp_ref, ht1_ref, dinv_ref, b1_ref, w2_ref, out_ref):
        acc = accp_ref[0] + accp_ref[1] + ht1_ref[...]
        z = jnp.maximum(acc * dinv_ref[...] + b1_ref[...], 0.0)
        h2 = jnp.dot(z, w2_ref[...], preferred_element_type=jnp.float32)
        out_ref[...] = h2 * dinv_ref[...]

    return pl.pallas_call(
        body,
        grid=(n // br,),
        in_specs=[
            pl.BlockSpec((NC, br, d), lambda i: (0, i, 0)),
            pl.BlockSpec((br, d), lambda i: (i, 0)),
            pl.BlockSpec((br, 1), lambda i: (i, 0)),
            pl.BlockSpec((1, d), lambda i: (0, 0)),
            pl.BlockSpec((d, d), lambda i: (0, 0)),
        ],
        out_specs=pl.BlockSpec((br, d), lambda i: (i, 0)),
        out_shape=jax.ShapeDtypeStruct((n, d), jnp.float32),
    )(accp, ht1, dinv, b1, w2)


def _tc_final(accp2, ht2, dinv, b2, br):
    n, d = ht2.shape
    inv_n = 1.0 / n

    def body(accp_ref, ht2_ref, dinv_ref, b2_ref, out_ref):
        i = pl.program_id(0)
        acc = accp_ref[0] + accp_ref[1] + ht2_ref[...]
        z = jnp.maximum(acc * dinv_ref[...] + b2_ref[...], 0.0)

        @pl.when(i == 0)
        def _():
            out_ref[...] = jnp.zeros_like(out_ref)

        out_ref[...] += jnp.sum(z, axis=0, keepdims=True)

        @pl.when(i == pl.num_programs(0) - 1)
        def _():
            out_ref[...] *= inv_n

    return pl.pallas_call(
        body,
        grid=(n // br,),
        in_specs=[
            pl.BlockSpec((NC, br, d), lambda i: (0, i, 0)),
            pl.BlockSpec((br, d), lambda i: (i, 0)),
            pl.BlockSpec((br, 1), lambda i: (i, 0)),
            pl.BlockSpec((1, d), lambda i: (0, 0)),
        ],
        out_specs=pl.BlockSpec((1, d), lambda i: (0, 0)),
        out_shape=jax.ShapeDtypeStruct((1, d), jnp.float32),
    )(accp2, ht2, dinv, b2)


def kernel(x, edge_index, W1, b1, W2, b2):
    n, d = x.shape
    e = edge_index.shape[1]

    np_deg = ((n + NS * CH - 1) // (NS * CH)) * (NS * CH)
    if np_deg == n:
        np_deg += NS * CH
    nps_deg = np_deg // NS
    np_acc = ((n + NS * 8 - 1) // (NS * 8)) * (NS * 8)
    if np_acc == n:
        np_acc += NS * 8
    nps_acc = np_acc // NS

    epw = ((e + NW * 8 * CH - 1) // (NW * 8 * CH)) * 8 * CH
    e_pad = epw * NW
    rpw = epw // CH

    src = edge_index[0].astype(jnp.int32)
    dst = edge_index[1].astype(jnp.int32)
    pad = e_pad - e
    src_p = jnp.concatenate([src, jnp.zeros((pad,), jnp.int32)])
    dst_p = jnp.concatenate([dst, jnp.full((pad,), n, jnp.int32)])
    src2d = src_p.reshape(NW * rpw, CH)
    dst2d = dst_p.reshape(NW * rpw, CH)

    zeros1 = jnp.zeros((nps_deg,), jnp.float32)
    zeros2 = jnp.zeros((nps_acc, d), jnp.float32)

    br = 1000 if n % 1000 == 0 else 8 * (n // 8)

    tot = (NW * rpw) // NS
    blk0 = tot

    degp = _sc_deg(dst2d, zeros1, np_deg, nps_deg)
    degp = degp.reshape(NC, np_deg)[:, :, None]
    ht1, dinv = _tc_scale1(x, W1, degp, br)
    accp1 = _sc_agg(ht1, src2d, dst2d, zeros2, np_acc, nps_acc, blk0)
    accp1 = accp1.reshape(NC, np_acc, d)
    ht2 = _tc_mid(accp1, ht1, dinv, b1.reshape(1, d), W2, br)
    accp2 = _sc_agg(ht2, src2d, dst2d, zeros2, np_acc, nps_acc, blk0)
    accp2 = accp2.reshape(NC, np_acc, d)
    return _tc_final(accp2, ht2, dinv, b2.reshape(1, d), br)


# speedup vs baseline: 4.0474x; 4.0474x over previous
import functools

import jax
import jax.numpy as jnp
from jax import lax
from jax.experimental import pallas as pl
from jax.experimental.pallas import tpu as pltpu
from jax.experimental.pallas import tpu_sc as plsc

NC = 2
NS = 16
NW = NC * NS
CH = 128


def _sc_deg(dst2d, zeros1, np_, nps):
    rpw = dst2d.shape[0] // NW
    mesh = plsc.VectorSubcoreMesh(core_axis_name="c", subcore_axis_name="s")

    @functools.partial(
        pl.kernel,
        out_type=jax.ShapeDtypeStruct((NC, NS, nps), jnp.float32),
        mesh=mesh,
        scratch_types=[
            pltpu.VMEM((rpw, CH), jnp.int32),
            pltpu.VMEM((CH,), jnp.float32),
            pltpu.VMEM_SHARED((np_,), jnp.float32),
        ],
    )
    def k(dst_hbm, z_hbm, out_hbm, idx_v, ones_v, deg_sh):
        cid = lax.axis_index("c")
        sid = lax.axis_index("s")
        wid = sid * NC + cid
        pltpu.sync_copy(z_hbm, deg_sh.at[pl.ds(sid * nps, nps)])

        @pl.loop(0, CH, step=16)
        def _(i):
            ones_v[pl.ds(i, 16)] = jnp.ones((16,), jnp.float32)

        pltpu.sync_copy(dst_hbm.at[pl.ds(wid * rpw, rpw)], idx_v)
        plsc.subcore_barrier()

        @pl.loop(0, rpw)
        def _(j):
            pltpu.sync_copy(ones_v, deg_sh.at[idx_v.at[j]], add=True)

        plsc.subcore_barrier()
        pltpu.sync_copy(deg_sh.at[pl.ds(sid * nps, nps)], out_hbm.at[cid, sid])

    return k(dst2d, zeros1)


CB = 48


def _sc_agg(h, src2d, dst2d, zeros2, np_, nps, b0):
    tblk = src2d.shape[0]
    tot = tblk // NS
    b1 = tot - b0
    d = h.shape[1]
    assert b0 % 8 == 0 and b1 % 8 == 0 and CB % 8 == 0
    mesh = plsc.VectorSubcoreMesh(core_axis_name="c", subcore_axis_name="s")

    nbuf = 2

    @functools.partial(
        pl.kernel,
        out_type=jax.ShapeDtypeStruct((NC, NS, nps, d), jnp.float32),
        mesh=mesh,
        scratch_types=[
            pltpu.VMEM((CB, CH), jnp.int32),
            pltpu.VMEM((CB, CH), jnp.int32),
            pltpu.VMEM((CH, d), jnp.float32),
            pltpu.VMEM((CH, d), jnp.float32),
            pltpu.VMEM_SHARED((np_, d), jnp.float32),
            pltpu.SemaphoreType.DMA,
            pltpu.SemaphoreType.DMA,
        ],
    )
    def k(h_hbm, s_hbm, d_hbm, z_hbm, out_hbm, src_v, dst_v, r0, r1,
          acc_sh, sem0, sem1):
        sems = [sem0, sem1]
        rows = [r0, r1]
        cid = lax.axis_index("c")
        sid = lax.axis_index("s")
        with jax.named_scope("agg_zero"):
            pltpu.sync_copy(z_hbm, acc_sh.at[pl.ds(sid * nps, nps)])
            plsc.subcore_barrier()

        def run(nblk, base):
            done = 0
            while done < nblk:
                nb = min(CB, nblk - done)
                cbase = base + done
                done += nb
                pltpu.sync_copy(s_hbm.at[pl.ds(cbase, nb)], src_v.at[pl.ds(0, nb)])
                pltpu.sync_copy(d_hbm.at[pl.ds(cbase, nb)], dst_v.at[pl.ds(0, nb)])

                for b in range(nbuf):
                    pltpu.async_copy(h_hbm.at[src_v.at[b]], rows[b], sems[b])

                @pl.loop(0, nb - nbuf, step=nbuf)
                def _(j):
                    for b in range(nbuf):
                        jj = j + b
                        pltpu.make_async_copy(
                            h_hbm.at[src_v.at[jj]], rows[b], sems[b]).wait()
                        pltpu.sync_copy(rows[b], acc_sh.at[dst_v.at[jj]],
                                        add=True)
                        pltpu.async_copy(
                            h_hbm.at[src_v.at[jj + nbuf]], rows[b], sems[b])

                for b in range(nbuf):
                    jj = nb - nbuf + b
                    pltpu.make_async_copy(
                        h_hbm.at[src_v.at[jj]], rows[b], sems[b]).wait()
                    pltpu.sync_copy(rows[b], acc_sh.at[dst_v.at[jj]],
                                    add=True)

        with jax.named_scope("agg_edges"):
            @pl.when(cid == 0)
            def _():
                run(b0, sid * b0)

            @pl.when(cid == 1)
            def _():
                run(b1, NS * b0 + sid * b1)

            plsc.subcore_barrier()

        with jax.named_scope("agg_dump"):
            pltpu.sync_copy(acc_sh.at[pl.ds(sid * nps, nps)],
                            out_hbm.at[cid, sid])

    return k(h, src2d, dst2d, zeros2)


def _tc_scale1(x, w1, degp, br):
    n, d = x.shape

    def body(x_ref, w_ref, degp_ref, ht_ref, dinv_ref):
        h = jnp.dot(x_ref[...], w_ref[...], preferred_element_type=jnp.float32)
        deg = degp_ref[0] + degp_ref[1] + 1.0
        dinv = lax.rsqrt(deg)
        dinv_ref[...] = dinv
        ht_ref[...] = h * dinv

    return pl.pallas_call(
        body,
        grid=(n // br,),
        in_specs=[
            pl.BlockSpec((br, d), lambda i: (i, 0)),
            pl.BlockSpec((d, d), lambda i: (0, 0)),
            pl.BlockSpec((NC, br, 1), lambda i: (0, i, 0)),
        ],
        out_specs=[
            pl.BlockSpec((br, d), lambda i: (i, 0)),
            pl.BlockSpec((br, 1), lambda i: (i, 0)),
        ],
        out_shape=[
            jax.ShapeDtypeStruct((n, d), jnp.float32),
            jax.ShapeDtypeStruct((n, 1), jnp.float32),
        ],
    )(x, w1, degp)


def _tc_mid(accp, ht1, dinv, b1, w2, br):
    n, d = ht1.shape

    def body(accp_ref, ht1_ref, dinv_ref, b1_ref, w2_ref, out_ref):
        acc = accp_ref[0] + accp_ref[1] + ht1_ref[...]
        z = jnp.maximum(acc * dinv_ref[...] + b1_ref[...], 0.0)
        h2 = jnp.dot(z, w2_ref[...], preferred_element_type=jnp.float32)
        out_ref[...] = h2 * dinv_ref[...]

    return pl.pallas_call(
        body,
        grid=(n // br,),
        in_specs=[
            pl.BlockSpec((NC, br, d), lambda i: (0, i, 0)),
            pl.BlockSpec((br, d), lambda i: (i, 0)),
            pl.BlockSpec((br, 1), lambda i: (i, 0)),
            pl.BlockSpec((1, d), lambda i: (0, 0)),
            pl.BlockSpec((d, d), lambda i: (0, 0)),
        ],
        out_specs=pl.BlockSpec((br, d), lambda i: (i, 0)),
        out_shape=jax.ShapeDtypeStruct((n, d), jnp.float32),
    )(accp, ht1, dinv, b1, w2)


def _tc_final(accp2, ht2, dinv, b2, br):
    n, d = ht2.shape
    inv_n = 1.0 / n

    def body(accp_ref, ht2_ref, dinv_ref, b2_ref, out_ref):
        i = pl.program_id(0)
        acc = accp_ref[0] + accp_ref[1] + ht2_ref[...]
        z = jnp.maximum(acc * dinv_ref[...] + b2_ref[...], 0.0)

        @pl.when(i == 0)
        def _():
            out_ref[...] = jnp.zeros_like(out_ref)

        out_ref[...] += jnp.sum(z, axis=0, keepdims=True)

        @pl.when(i == pl.num_programs(0) - 1)
        def _():
            out_ref[...] *= inv_n

    return pl.pallas_call(
        body,
        grid=(n // br,),
        in_specs=[
            pl.BlockSpec((NC, br, d), lambda i: (0, i, 0)),
            pl.BlockSpec((br, d), lambda i: (i, 0)),
            pl.BlockSpec((br, 1), lambda i: (i, 0)),
            pl.BlockSpec((1, d), lambda i: (0, 0)),
        ],
        out_specs=pl.BlockSpec((1, d), lambda i: (0, 0)),
        out_shape=jax.ShapeDtypeStruct((1, d), jnp.float32),
    )(accp2, ht2, dinv, b2)


def kernel(x, edge_index, W1, b1, W2, b2):
    n, d = x.shape
    e = edge_index.shape[1]

    np_deg = ((n + NS * CH - 1) // (NS * CH)) * (NS * CH)
    if np_deg == n:
        np_deg += NS * CH
    nps_deg = np_deg // NS
    np_acc = ((n + NS * 8 - 1) // (NS * 8)) * (NS * 8)
    if np_acc == n:
        np_acc += NS * 8
    nps_acc = np_acc // NS

    epw = ((e + NW * 8 * CH - 1) // (NW * 8 * CH)) * 8 * CH
    e_pad = epw * NW
    rpw = epw // CH

    src = edge_index[0].astype(jnp.int32)
    dst = edge_index[1].astype(jnp.int32)
    pad = e_pad - e
    pad_i = jnp.arange(pad, dtype=jnp.int32)
    src_p = jnp.concatenate([src, pad_i % n])
    dst_p = jnp.concatenate([dst, n + pad_i % (np_acc - n)])
    src2d = src_p.reshape(NW * rpw, CH)
    dst2d = dst_p.reshape(NW * rpw, CH)

    zeros1 = jnp.zeros((nps_deg,), jnp.float32)
    zeros2 = jnp.zeros((nps_acc, d), jnp.float32)

    br = 1000 if n % 1000 == 0 else 8 * (n // 8)

    tot = (NW * rpw) // NS
    blk0 = (tot // 2 + 7) // 8 * 8

    degp = _sc_deg(dst2d, zeros1, np_deg, nps_deg)
    degp = degp.reshape(NC, np_deg)[:, :, None]
    ht1, dinv = _tc_scale1(x, W1, degp, br)
    accp1 = _sc_agg(ht1, src2d, dst2d, zeros2, np_acc, nps_acc, blk0)
    accp1 = accp1.reshape(NC, np_acc, d)
    ht2 = _tc_mid(accp1, ht1, dinv, b1.reshape(1, d), W2, br)
    accp2 = _sc_agg(ht2, src2d, dst2d, zeros2, np_acc, nps_acc, blk0)
    accp2 = accp2.reshape(NC, np_acc, d)
    return _tc_final(accp2, ht2, dinv, b2.reshape(1, d), br)


# speedup vs baseline: 4.1279x; 1.0199x over previous
import functools

import jax
import jax.numpy as jnp
from jax import lax
from jax.experimental import pallas as pl
from jax.experimental.pallas import tpu as pltpu
from jax.experimental.pallas import tpu_sc as plsc

NC = 2
NS = 16
NW = NC * NS
CH = 128


def _sc_deg(dst2d, zeros1, np_, nps):
    rpw = dst2d.shape[0] // NW
    mesh = plsc.VectorSubcoreMesh(core_axis_name="c", subcore_axis_name="s")

    @functools.partial(
        pl.kernel,
        out_type=jax.ShapeDtypeStruct((NC, NS, nps), jnp.float32),
        mesh=mesh,
        scratch_types=[
            pltpu.VMEM((rpw, CH), jnp.int32),
            pltpu.VMEM((CH,), jnp.float32),
            pltpu.VMEM_SHARED((np_,), jnp.float32),
        ],
    )
    def k(dst_hbm, z_hbm, out_hbm, idx_v, ones_v, deg_sh):
        cid = lax.axis_index("c")
        sid = lax.axis_index("s")
        wid = sid * NC + cid
        pltpu.sync_copy(z_hbm, deg_sh.at[pl.ds(sid * nps, nps)])

        @pl.loop(0, CH, step=16)
        def _(i):
            ones_v[pl.ds(i, 16)] = jnp.ones((16,), jnp.float32)

        pltpu.sync_copy(dst_hbm.at[pl.ds(wid * rpw, rpw)], idx_v)
        plsc.subcore_barrier()

        @pl.loop(0, rpw)
        def _(j):
            pltpu.sync_copy(ones_v, deg_sh.at[idx_v.at[j]], add=True)

        plsc.subcore_barrier()
        pltpu.sync_copy(deg_sh.at[pl.ds(sid * nps, nps)], out_hbm.at[cid, sid])

    return k(dst2d, zeros1)


CB = 48


def _sc_agg(h, src2d, dst2d, zeros2, np_, nps, b0):
    tblk = src2d.shape[0]
    tot = tblk // NS
    b1 = tot - b0
    d = h.shape[1]
    assert b0 % 8 == 0 and b1 % 8 == 0 and CB % 8 == 0
    mesh = plsc.VectorSubcoreMesh(core_axis_name="c", subcore_axis_name="s")

    nbuf = 2

    @functools.partial(
        pl.kernel,
        out_type=jax.ShapeDtypeStruct((NC, NS, nps, d), jnp.float32),
        mesh=mesh,
        scratch_types=[
            pltpu.VMEM((CB, CH), jnp.int32),
            pltpu.VMEM((CB, CH), jnp.int32),
            pltpu.VMEM((CH, d), jnp.float32),
            pltpu.VMEM((CH, d), jnp.float32),
            pltpu.VMEM_SHARED((np_, d), jnp.float32),
            pltpu.SemaphoreType.DMA,
            pltpu.SemaphoreType.DMA,
        ],
    )
    def k(h_hbm, s_hbm, d_hbm, z_hbm, out_hbm, src_v, dst_v, r0, r1,
          acc_sh, sem0, sem1):
        sems = [sem0, sem1]
        rows = [r0, r1]
        cid = lax.axis_index("c")
        sid = lax.axis_index("s")
        with jax.named_scope("agg_zero"):
            pltpu.sync_copy(z_hbm, acc_sh.at[pl.ds(sid * nps, nps)])
            plsc.subcore_barrier()

        def run(nblk, base):
            done = 0
            while done < nblk:
                nb = min(CB, nblk - done)
                cbase = base + done
                done += nb
                pltpu.sync_copy(s_hbm.at[pl.ds(cbase, nb)], src_v.at[pl.ds(0, nb)])
                pltpu.sync_copy(d_hbm.at[pl.ds(cbase, nb)], dst_v.at[pl.ds(0, nb)])

                for b in range(nbuf):
                    pltpu.async_copy(h_hbm.at[src_v.at[b]], rows[b], sems[b])

                @pl.loop(0, nb - nbuf, step=nbuf)
                def _(j):
                    for b in range(nbuf):
                        jj = j + b
                        pltpu.make_async_copy(
                            h_hbm.at[src_v.at[jj]], rows[b], sems[b]).wait()
                        pltpu.sync_copy(rows[b], acc_sh.at[dst_v.at[jj]],
                                        add=True)
                        pltpu.async_copy(
                            h_hbm.at[src_v.at[jj + nbuf]], rows[b], sems[b])

                for b in range(nbuf):
                    jj = nb - nbuf + b
                    pltpu.make_async_copy(
                        h_hbm.at[src_v.at[jj]], rows[b], sems[b]).wait()
                    pltpu.sync_copy(rows[b], acc_sh.at[dst_v.at[jj]],
                                    add=True)

        with jax.named_scope("agg_edges"):
            @pl.when(cid == 0)
            def _():
                run(b0, sid * b0)

            @pl.when(cid == 1)
            def _():
                run(b1, NS * b0 + sid * b1)

            plsc.subcore_barrier()

        with jax.named_scope("agg_dump"):
            pltpu.sync_copy(acc_sh.at[pl.ds(sid * nps, nps)],
                            out_hbm.at[cid, sid])

    return k(h, src2d, dst2d, zeros2)


def _tc_mm1(x, w1, br):
    n, d = x.shape

    def body(x_ref, w_ref, out_ref):
        out_ref[...] = jnp.dot(x_ref[...], w_ref[...],
                               preferred_element_type=jnp.float32)

    return pl.pallas_call(
        body,
        grid=(n // br,),
        in_specs=[
            pl.BlockSpec((br, d), lambda i: (i, 0)),
            pl.BlockSpec((d, d), lambda i: (0, 0)),
        ],
        out_specs=pl.BlockSpec((br, d), lambda i: (i, 0)),
        out_shape=jax.ShapeDtypeStruct((n, d), jnp.float32),
    )(x, w1)


def _tc_scale1(h1, degp, br):
    n, d = h1.shape

    def body(h_ref, degp_ref, ht_ref, dinv_ref):
        deg = degp_ref[0] + degp_ref[1] + 1.0
        dinv = lax.rsqrt(deg)
        dinv_ref[...] = dinv
        ht_ref[...] = h_ref[...] * dinv

    return pl.pallas_call(
        body,
        grid=(n // br,),
        in_specs=[
            pl.BlockSpec((br, d), lambda i: (i, 0)),
            pl.BlockSpec((NC, br, 1), lambda i: (0, i, 0)),
        ],
        out_specs=[
            pl.BlockSpec((br, d), lambda i: (i, 0)),
            pl.BlockSpec((br, 1), lambda i: (i, 0)),
        ],
        out_shape=[
            jax.ShapeDtypeStruct((n, d), jnp.float32),
            jax.ShapeDtypeStruct((n, 1), jnp.float32),
        ],
    )(h1, degp)


def _tc_mid(accp, ht1, dinv, b1, w2, br):
    n, d = ht1.shape

    def body(accp_ref, ht1_ref, dinv_ref, b1_ref, w2_ref, out_ref):
        acc = accp_ref[0] + accp_ref[1] + ht1_ref[...]
        z = jnp.maximum(acc * dinv_ref[...] + b1_ref[...], 0.0)
        h2 = jnp.dot(z, w2_ref[...], preferred_element_type=jnp.float32)
        out_ref[...] = h2 * dinv_ref[...]

    return pl.pallas_call(
        body,
        grid=(n // br,),
        in_specs=[
            pl.BlockSpec((NC, br, d), lambda i: (0, i, 0)),
            pl.BlockSpec((br, d), lambda i: (i, 0)),
            pl.BlockSpec((br, 1), lambda i: (i, 0)),
            pl.BlockSpec((1, d), lambda i: (0, 0)),
            pl.BlockSpec((d, d), lambda i: (0, 0)),
        ],
        out_specs=pl.BlockSpec((br, d), lambda i: (i, 0)),
        out_shape=jax.ShapeDtypeStruct((n, d), jnp.float32),
    )(accp, ht1, dinv, b1, w2)


def _tc_final(accp2, ht2, dinv, b2, br):
    n, d = ht2.shape
    inv_n = 1.0 / n

    def body(accp_ref, ht2_ref, dinv_ref, b2_ref, out_ref):
        i = pl.program_id(0)
        acc = accp_ref[0] + accp_ref[1] + ht2_ref[...]
        z = jnp.maximum(acc * dinv_ref[...] + b2_ref[...], 0.0)

        @pl.when(i == 0)
        def _():
            out_ref[...] = jnp.zeros_like(out_ref)

        out_ref[...] += jnp.sum(z, axis=0, keepdims=True)

        @pl.when(i == pl.num_programs(0) - 1)
        def _():
            out_ref[...] *= inv_n

    return pl.pallas_call(
        body,
        grid=(n // br,),
        in_specs=[
            pl.BlockSpec((NC, br, d), lambda i: (0, i, 0)),
            pl.BlockSpec((br, d), lambda i: (i, 0)),
            pl.BlockSpec((br, 1), lambda i: (i, 0)),
            pl.BlockSpec((1, d), lambda i: (0, 0)),
        ],
        out_specs=pl.BlockSpec((1, d), lambda i: (0, 0)),
        out_shape=jax.ShapeDtypeStruct((1, d), jnp.float32),
    )(accp2, ht2, dinv, b2)


def kernel(x, edge_index, W1, b1, W2, b2):
    n, d = x.shape
    e = edge_index.shape[1]

    np_deg = ((n + NS * CH - 1) // (NS * CH)) * (NS * CH)
    if np_deg == n:
        np_deg += NS * CH
    nps_deg = np_deg // NS
    np_acc = ((n + NS * 8 - 1) // (NS * 8)) * (NS * 8)
    if np_acc == n:
        np_acc += NS * 8
    nps_acc = np_acc // NS

    epw = ((e + NW * 8 * CH - 1) // (NW * 8 * CH)) * 8 * CH
    e_pad = epw * NW
    rpw = epw // CH

    src = edge_index[0].astype(jnp.int32)
    dst = edge_index[1].astype(jnp.int32)
    pad = e_pad - e
    pad_i = jnp.arange(pad, dtype=jnp.int32)
    src_p = jnp.concatenate([src, pad_i % n])
    dst_p = jnp.concatenate([dst, n + pad_i % (np_acc - n)])
    src2d = src_p.reshape(NW * rpw, CH)
    dst2d = dst_p.reshape(NW * rpw, CH)

    zeros1 = jnp.zeros((nps_deg,), jnp.float32)
    zeros2 = jnp.zeros((nps_acc, d), jnp.float32)

    br = n
    for cand in (2000, 1600, 1040, 400, 80, 16):
        if n % cand == 0:
            br = cand
            break

    tot = (NW * rpw) // NS
    blk0 = (tot // 2 + 7) // 8 * 8

    h1 = _tc_mm1(x, W1, br)
    degp = _sc_deg(dst2d, zeros1, np_deg, nps_deg)
    degp = degp.reshape(NC, np_deg)[:, :, None]
    ht1, dinv = _tc_scale1(h1, degp, br)
    accp1 = _sc_agg(ht1, src2d, dst2d, zeros2, np_acc, nps_acc, blk0)
    accp1 = accp1.reshape(NC, np_acc, d)
    ht2 = _tc_mid(accp1, ht1, dinv, b1.reshape(1, d), W2, br)
    accp2 = _sc_agg(ht2, src2d, dst2d, zeros2, np_acc, nps_acc, blk0)
    accp2 = accp2.reshape(NC, np_acc, d)
    return _tc_final(accp2, ht2, dinv, b2.reshape(1, d), br)


# speedup vs baseline: 4.2874x; 1.0386x over previous
import functools

import jax
import jax.numpy as jnp
from jax import lax
from jax.experimental import pallas as pl
from jax.experimental.pallas import tpu as pltpu
from jax.experimental.pallas import tpu_sc as plsc

NC = 2
NS = 16
NW = NC * NS
CH = 128


def _sc_deg(dst2d, zeros1, np_, nps):
    rpw = dst2d.shape[0] // NW
    mesh = plsc.VectorSubcoreMesh(core_axis_name="c", subcore_axis_name="s")

    @functools.partial(
        pl.kernel,
        out_type=jax.ShapeDtypeStruct((NC, NS, nps), jnp.float32),
        mesh=mesh,
        scratch_types=[
            pltpu.VMEM((rpw, CH), jnp.int32),
            pltpu.VMEM((CH,), jnp.float32),
            pltpu.VMEM_SHARED((np_,), jnp.float32),
        ],
    )
    def k(dst_hbm, z_hbm, out_hbm, idx_v, ones_v, deg_sh):
        cid = lax.axis_index("c")
        sid = lax.axis_index("s")
        wid = sid * NC + cid
        pltpu.sync_copy(z_hbm, deg_sh.at[pl.ds(sid * nps, nps)])

        @pl.loop(0, CH, step=16)
        def _(i):
            ones_v[pl.ds(i, 16)] = jnp.ones((16,), jnp.float32)

        pltpu.sync_copy(dst_hbm.at[pl.ds(wid * rpw, rpw)], idx_v)
        plsc.subcore_barrier()

        @pl.loop(0, rpw)
        def _(j):
            pltpu.sync_copy(ones_v, deg_sh.at[idx_v.at[j]], add=True)

        plsc.subcore_barrier()
        pltpu.sync_copy(deg_sh.at[pl.ds(sid * nps, nps)], out_hbm.at[cid, sid])

    return k(dst2d, zeros1)


CB = 32


def _sc_agg(h, src2d, dst2d, np_, nps, b0):
    tblk = src2d.shape[0]
    tot = tblk // NS
    b1 = tot - b0
    d = h.shape[1]
    assert b0 % 8 == 0 and b1 % 8 == 0 and CB % 8 == 0
    mesh = plsc.VectorSubcoreMesh(core_axis_name="c", subcore_axis_name="s")

    nbuf = 2

    @functools.partial(
        pl.kernel,
        out_type=jax.ShapeDtypeStruct((NC, NS, nps, d), jnp.float32),
        mesh=mesh,
        scratch_types=[
            pltpu.VMEM((2, CB, CH), jnp.int32),
            pltpu.VMEM((2, CB, CH), jnp.int32),
            pltpu.VMEM((CH, d), jnp.float32),
            pltpu.VMEM((CH, d), jnp.float32),
            pltpu.VMEM_SHARED((np_, d), jnp.float32),
            pltpu.SemaphoreType.DMA,
            pltpu.SemaphoreType.DMA,
            pltpu.SemaphoreType.DMA,
            pltpu.SemaphoreType.DMA,
        ],
    )
    def k(h_hbm, s_hbm, d_hbm, out_hbm, src_v, dst_v, r0, r1,
          acc_sh, sem0, sem1, isem0, isem1):
        sems = [sem0, sem1]
        isems = [isem0, isem1]
        rows = [r0, r1]
        cid = lax.axis_index("c")
        sid = lax.axis_index("s")

        with jax.named_scope("agg_zero"):
            @pl.loop(0, CH)
            def _(r):
                @pl.loop(0, d, step=16)
                def _(c):
                    r0[r, pl.ds(c, 16)] = jnp.zeros((16,), jnp.float32)

            @pl.loop(0, nps - CH, step=CH)
            def _(r):
                pltpu.sync_copy(r0, acc_sh.at[pl.ds(sid * nps + r, CH)])
            rem = nps % CH if nps % CH else CH
            pltpu.sync_copy(
                r0.at[pl.ds(0, rem)],
                acc_sh.at[pl.ds(sid * nps + nps - rem, rem)])
            plsc.subcore_barrier()

        def run(nblk, base):
            nchunks = (nblk + CB - 1) // CB

            def load_idx(c, buf):
                nb = min(CB, nblk - c * CB)
                cbase = base + c * CB
                pltpu.async_copy(s_hbm.at[pl.ds(cbase, nb)],
                                 src_v.at[buf].at[pl.ds(0, nb)], isems[0])
                pltpu.async_copy(d_hbm.at[pl.ds(cbase, nb)],
                                 dst_v.at[buf].at[pl.ds(0, nb)], isems[1])

            def wait_idx(c, buf):
                nb = min(CB, nblk - c * CB)
                cbase = base + c * CB
                pltpu.make_async_copy(
                    s_hbm.at[pl.ds(cbase, nb)],
                    src_v.at[buf].at[pl.ds(0, nb)], isems[0]).wait()
                pltpu.make_async_copy(
                    d_hbm.at[pl.ds(cbase, nb)],
                    dst_v.at[buf].at[pl.ds(0, nb)], isems[1]).wait()

            load_idx(0, 0)
            for c in range(nchunks):
                cur = c % 2
                nb = min(CB, nblk - c * CB)
                wait_idx(c, cur)
                if c + 1 < nchunks:
                    load_idx(c + 1, 1 - cur)
                sv = src_v.at[cur]
                dv = dst_v.at[cur]

                for b in range(nbuf):
                    pltpu.async_copy(h_hbm.at[sv.at[b]], rows[b], sems[b])

                @pl.loop(0, nb - nbuf, step=nbuf)
                def _(j):
                    for b in range(nbuf):
                        jj = j + b
                        pltpu.make_async_copy(
                            h_hbm.at[sv.at[jj]], rows[b], sems[b]).wait()
                        pltpu.sync_copy(rows[b], acc_sh.at[dv.at[jj]],
                                        add=True)
                        pltpu.async_copy(
                            h_hbm.at[sv.at[jj + nbuf]], rows[b], sems[b])

                for b in range(nbuf):
                    jj = nb - nbuf + b
                    pltpu.make_async_copy(
                        h_hbm.at[sv.at[jj]], rows[b], sems[b]).wait()
                    pltpu.sync_copy(rows[b], acc_sh.at[dv.at[jj]],
                                    add=True)

        with jax.named_scope("agg_edges"):
            @pl.when(cid == 0)
            def _():
                run(b0, sid * b0)

            @pl.when(cid == 1)
            def _():
                run(b1, NS * b0 + sid * b1)

            plsc.subcore_barrier()

        with jax.named_scope("agg_dump"):
            pltpu.sync_copy(acc_sh.at[pl.ds(sid * nps, nps)],
                            out_hbm.at[cid, sid])

    return k(h, src2d, dst2d)


def _tc_mm1(x, w1, br):
    n, d = x.shape

    def body(x_ref, w_ref, out_ref):
        out_ref[...] = jnp.dot(x_ref[...], w_ref[...],
                               preferred_element_type=jnp.float32)

    return pl.pallas_call(
        body,
        grid=(n // br,),
        in_specs=[
            pl.BlockSpec((br, d), lambda i: (i, 0)),
            pl.BlockSpec((d, d), lambda i: (0, 0)),
        ],
        out_specs=pl.BlockSpec((br, d), lambda i: (i, 0)),
        out_shape=jax.ShapeDtypeStruct((n, d), jnp.float32),
    )(x, w1)


def _tc_scale1(h1, degp, br):
    n, d = h1.shape

    def body(h_ref, degp_ref, ht_ref, dinv_ref):
        deg = degp_ref[0] + degp_ref[1] + 1.0
        dinv = lax.rsqrt(deg)
        dinv_ref[...] = dinv
        ht_ref[...] = h_ref[...] * dinv

    return pl.pallas_call(
        body,
        grid=(n // br,),
        in_specs=[
            pl.BlockSpec((br, d), lambda i: (i, 0)),
            pl.BlockSpec((NC, br, 1), lambda i: (0, i, 0)),
        ],
        out_specs=[
            pl.BlockSpec((br, d), lambda i: (i, 0)),
            pl.BlockSpec((br, 1), lambda i: (i, 0)),
        ],
        out_shape=[
            jax.ShapeDtypeStruct((n, d), jnp.float32),
            jax.ShapeDtypeStruct((n, 1), jnp.float32),
        ],
    )(h1, degp)


def _tc_mid(accp, ht1, dinv, b1, w2, br):
    n, d = ht1.shape

    def body(accp_ref, ht1_ref, dinv_ref, b1_ref, w2_ref, out_ref):
        acc = accp_ref[0] + accp_ref[1] + ht1_ref[...]
        z = jnp.maximum(acc * dinv_ref[...] + b1_ref[...], 0.0)
        h2 = jnp.dot(z, w2_ref[...], preferred_element_type=jnp.float32)
        out_ref[...] = h2 * dinv_ref[...]

    return pl.pallas_call(
        body,
        grid=(n // br,),
        in_specs=[
            pl.BlockSpec((NC, br, d), lambda i: (0, i, 0)),
            pl.BlockSpec((br, d), lambda i: (i, 0)),
            pl.BlockSpec((br, 1), lambda i: (i, 0)),
            pl.BlockSpec((1, d), lambda i: (0, 0)),
            pl.BlockSpec((d, d), lambda i: (0, 0)),
        ],
        out_specs=pl.BlockSpec((br, d), lambda i: (i, 0)),
        out_shape=jax.ShapeDtypeStruct((n, d), jnp.float32),
    )(accp, ht1, dinv, b1, w2)


def _tc_final(accp2, ht2, dinv, b2, br):
    n, d = ht2.shape
    inv_n = 1.0 / n

    def body(accp_ref, ht2_ref, dinv_ref, b2_ref, out_ref):
        i = pl.program_id(0)
        acc = accp_ref[0] + accp_ref[1] + ht2_ref[...]
        z = jnp.maximum(acc * dinv_ref[...] + b2_ref[...], 0.0)

        @pl.when(i == 0)
        def _():
            out_ref[...] = jnp.zeros_like(out_ref)

        out_ref[...] += jnp.sum(z, axis=0, keepdims=True)

        @pl.when(i == pl.num_programs(0) - 1)
        def _():
            out_ref[...] *= inv_n

    return pl.pallas_call(
        body,
        grid=(n // br,),
        in_specs=[
            pl.BlockSpec((NC, br, d), lambda i: (0, i, 0)),
            pl.BlockSpec((br, d), lambda i: (i, 0)),
            pl.BlockSpec((br, 1), lambda i: (i, 0)),
            pl.BlockSpec((1, d), lambda i: (0, 0)),
        ],
        out_specs=pl.BlockSpec((1, d), lambda i: (0, 0)),
        out_shape=jax.ShapeDtypeStruct((1, d), jnp.float32),
    )(accp2, ht2, dinv, b2)


def kernel(x, edge_index, W1, b1, W2, b2):
    n, d = x.shape
    e = edge_index.shape[1]

    np_deg = ((n + NS * CH - 1) // (NS * CH)) * (NS * CH)
    if np_deg == n:
        np_deg += NS * CH
    nps_deg = np_deg // NS
    np_acc = ((n + NS * 8 - 1) // (NS * 8)) * (NS * 8)
    if np_acc == n:
        np_acc += NS * 8
    nps_acc = np_acc // NS

    epw = ((e + NW * 8 * CH - 1) // (NW * 8 * CH)) * 8 * CH
    e_pad = epw * NW
    rpw = epw // CH

    src = edge_index[0].astype(jnp.int32)
    dst = edge_index[1].astype(jnp.int32)
    pad = e_pad - e
    pad_i = jnp.arange(pad, dtype=jnp.int32)
    src_p = jnp.concatenate([src, pad_i % n])
    dst_p = jnp.concatenate([dst, n + pad_i % (np_acc - n)])
    src2d = src_p.reshape(NW * rpw, CH)
    dst2d = dst_p.reshape(NW * rpw, CH)

    zeros1 = jnp.zeros((nps_deg,), jnp.float32)

    br = n
    for cand in (2000, 1600, 1040, 400, 80, 16):
        if n % cand == 0:
            br = cand
            break

    tot = (NW * rpw) // NS
    blk0 = (tot // 2 + 7) // 8 * 8

    h1 = _tc_mm1(x, W1, br)
    degp = _sc_deg(dst2d, zeros1, np_deg, nps_deg)
    degp = degp.reshape(NC, np_deg)[:, :, None]
    ht1, dinv = _tc_scale1(h1, degp, br)
    accp1 = _sc_agg(ht1, src2d, dst2d, np_acc, nps_acc, blk0)
    accp1 = accp1.reshape(NC, np_acc, d)
    ht2 = _tc_mid(accp1, ht1, dinv, b1.reshape(1, d), W2, br)
    accp2 = _sc_agg(ht2, src2d, dst2d, np_acc, nps_acc, blk0)
    accp2 = accp2.reshape(NC, np_acc, d)
    return _tc_final(accp2, ht2, dinv, b2.reshape(1, d), br)
